# baseline (speedup 1.0000x reference)
import functools

import jax
import jax.numpy as jnp
from jax import lax
from jax.experimental import pallas as pl
from jax.experimental.pallas import tpu as pltpu
from jax.experimental.pallas import tpu_sc as plsc

CH = 128
NBUF = 4
NSP = 2


@functools.lru_cache(maxsize=None)
def _make_gather(V, D, NROWS):
    info = plsc.get_sparse_core_info()
    NC, NS = info.num_cores, info.num_subcores
    NW = NC * NS
    assert NROWS % (CH * NW) == 0
    nch = NROWS // (CH * NW)
    assert nch % 2 == 0 and nch >= 8
    mesh = plsc.VectorSubcoreMesh(core_axis_name="c", subcore_axis_name="s")

    @functools.partial(
        pl.kernel,
        mesh=mesh,
        out_type=jax.ShapeDtypeStruct((NROWS, D), jnp.float32),
        scratch_types=[
            pltpu.VMEM((nch, CH), jnp.int32),
            pltpu.VMEM((NBUF, CH, D), jnp.float32),
            pltpu.VMEM_SHARED((NS, NSP, CH, D), jnp.float32),
            pltpu.SemaphoreType.DMA,
            pltpu.SemaphoreType.DMA,
            pltpu.SemaphoreType.DMA,
            pltpu.SemaphoreType.DMA,
        ],
    )
    def k(table_hbm, idx_hbm, out_hbm, idx_v, rows_v, spm, gsem, pdsem, s1, s2):
        wid = lax.axis_index("s") * NC + lax.axis_index("c")
        sid = lax.axis_index("s")
        pltpu.sync_copy(idx_hbm.at[pl.ds(wid * nch, nch)], idx_v)
        row_base = wid * nch * CH

        def g_start(t):
            pltpu.async_copy(
                table_hbm.at[idx_v.at[t]], rows_v.at[lax.rem(t, NBUF)], gsem
            )

        def g_wait():
            pltpu.make_async_copy(
                table_hbm.at[idx_v.at[0]], rows_v.at[0], gsem
            ).wait()

        def pd_start(t):
            pltpu.async_copy(
                rows_v.at[lax.rem(t, NBUF)],
                out_hbm.at[pl.ds(row_base + t * CH, CH)],
                pdsem,
            )

        def pd_wait():
            pltpu.make_async_copy(
                rows_v.at[0], out_hbm.at[pl.ds(row_base, CH)], pdsem
            ).wait()

        def l1_start(t):
            pltpu.async_copy(
                rows_v.at[lax.rem(t, NBUF)],
                spm.at[sid, lax.rem(lax.div(t, 2), NSP)],
                s1,
            )

        def l1_wait():
            pltpu.make_async_copy(rows_v.at[0], spm.at[sid, 0], s1).wait()

        def l2_start(t):
            pltpu.async_copy(
                spm.at[sid, lax.rem(lax.div(t, 2), NSP)],
                out_hbm.at[pl.ds(row_base + t * CH, CH)],
                s2,
            )

        def l2_wait():
            pltpu.make_async_copy(
                spm.at[sid, 0], out_hbm.at[pl.ds(row_base, CH)], s2
            ).wait()

        g_start(0)
        g_start(1)
        g_start(2)
        g_wait()
        pd_start(0)
        g_start(3)
        g_wait()
        l1_start(1)
        pd_wait()
        g_start(4)
        g_wait()
        pd_start(2)
        l1_wait()
        g_start(5)
        g_wait()
        l1_start(3)
        l2_start(1)

        def body(p, _):
            t = p * 2
            l2_wait()
            pd_wait()
            g_start(t + 2)
            g_wait()
            pd_start(t)
            l1_wait()
            g_start(t + 3)
            g_wait()
            l1_start(t + 1)
            l2_start(t - 1)
            return ()

        lax.fori_loop(2, nch // 2 - 1, body, (), unroll=False)

        t = nch - 2
        l2_wait()
        pd_wait()
        g_wait()
        pd_start(t)
        l1_wait()
        g_wait()
        l1_start(t + 1)
        l2_start(t - 1)
        l1_wait()
        l2_start(nch - 1)
        pd_wait()
        l2_wait()
        l2_wait()

    return k


def kernel(input, weight):
    B, S = input.shape
    V, D = weight.shape
    idx = input.reshape(-1).astype(jnp.int32).reshape(-1, CH)
    out = _make_gather(V, D, B * S)(weight, idx)
    return out.reshape(B, S, D)
